# baseline (speedup 1.0000x reference)
import numpy as np

import jax
import jax.numpy as jnp
from jax import lax
from jax.experimental import pallas as pl
from jax.experimental.pallas import tpu as pltpu

_TAPS = tuple((dy, dx) for dy in range(3) for dx in range(3))
_CONTRACT0 = (((0,), (0,)), ((), ()))


def _mega_body(x_ref, w1t_ref, b1r_ref, w2t_ref, b2r_ref, wl_ref, bl_ref,
               w1_ref, b1_ref, wf_ref, bf_ref, w4_ref, b4_ref, m_ref, t_ref,
               o_ref, zp_ref, cs_ref, p2_ref, pm_ref):
    f = w1t_ref.shape[1]
    x = x_ref[0]
    xb = x.astype(jnp.bfloat16)

    c1t = lax.dot_general(xb, w1t_ref[...], _CONTRACT0,
                          preferred_element_type=jnp.float32) + b1r_ref[...]

    zp_ref[0:1, :, :] = jnp.zeros((1, 66, f), jnp.float32)
    zp_ref[:, 0:1, :] = jnp.zeros((66, 1, f), jnp.float32)
    zp_ref[1:65, 1:65, :] = c1t.reshape(64, 64, f)
    for dx in range(3):
        cs_ref[dx] = pltpu.bitcast(
            zp_ref[:, dx:dx + 63:2, :].astype(jnp.bfloat16), jnp.int32)
    acc = jnp.zeros((1024, f), jnp.float32) + b2r_ref[...]
    for k, (dy, dx) in enumerate(_TAPS):
        tap = pltpu.bitcast(cs_ref[dx, dy:dy + 63:2, :, :],
                            jnp.bfloat16)
        acc = acc + jnp.dot(tap.reshape(1024, f), w2t_ref[k],
                            preferred_element_type=jnp.float32)
    p2_ref[...] = acc.reshape(32, 32, f)

    m1 = p2_ref[0:25:3, :, :]
    for ky in range(1, 7):
        m1 = jnp.maximum(m1, p2_ref[ky:ky + 25:3, :, :])
    pm_ref[...] = m1
    vm = pm_ref[:, 0:25:3, :]
    for kx in range(1, 7):
        vm = jnp.maximum(vm, pm_ref[:, kx:kx + 25:3, :])

    z = vm.reshape(81, f).astype(jnp.bfloat16)
    for layer in range(3):
        zs = jnp.dot(t_ref[...], z,
                     preferred_element_type=jnp.float32).astype(jnp.bfloat16)
        acc = jnp.zeros((81, f), jnp.float32) + bl_ref[layer]
        for k in range(9):
            acc = acc + jnp.dot(zs[88 * k:88 * k + 81, :], wl_ref[layer, k],
                                preferred_element_type=jnp.float32)
        if layer < 2:
            acc = jnp.maximum(acc, 0.0)
        z = acc.astype(jnp.bfloat16)
    c3t = z

    up = lax.dot_general(c3t.astype(jnp.bfloat16), m_ref[...], _CONTRACT0,
                         preferred_element_type=jnp.float32)

    c1 = jnp.dot(w1_ref[...], xb, preferred_element_type=jnp.float32) + b1_ref[...]
    cf = (jnp.dot(wf_ref[...], c1.astype(jnp.bfloat16),
                  preferred_element_type=jnp.float32) + bf_ref[...] + up)
    c4 = jnp.dot(w4_ref[...], cf.astype(jnp.bfloat16),
                 preferred_element_type=jnp.float32) + b4_ref[...]
    o_ref[0] = x * jax.nn.sigmoid(c4)


def _shift_gather(hm, wm):
    hw = hm * wm
    t = np.zeros((9 * 88, hw), np.float32)
    for k, (dy, dx) in enumerate(_TAPS):
        for i in range(hm):
            si = i + dy - 1
            if si < 0 or si >= hm:
                continue
            for j in range(wm):
                sj = j + dx - 1
                if 0 <= sj < wm:
                    t[88 * k + i * wm + j, si * wm + sj] = 1.0
    return t


def _bilinear_matrix(out_size, in_size):
    scale = in_size / out_size
    dst = np.arange(out_size, dtype=np.float64)
    src = np.clip((dst + 0.5) * scale - 0.5, 0.0, in_size - 1)
    i0 = np.clip(np.floor(src).astype(np.int64), 0, in_size - 1)
    i1 = np.minimum(i0 + 1, in_size - 1)
    w1 = (src - i0).astype(np.float32)
    w0 = 1.0 - w1
    m = np.zeros((out_size, in_size), np.float32)
    rows = np.arange(out_size)
    np.add.at(m, (rows, i0), w0)
    np.add.at(m, (rows, i1), w1)
    return m


def kernel(x, w1, b1, wf, bf, w_max, b_max, w2, b2, w3, b3, w3_, b3_, w4, b4):
    B, C, H, W = x.shape
    N = H * W
    f = w1.shape[0]
    x_flat = x.reshape(B, C, N)
    bh = jnp.bfloat16

    def ktaps(w):
        return jnp.transpose(w, (2, 3, 1, 0)).reshape(9, f, f)

    w1t = jnp.transpose(w1[:, :, 0, 0])
    wl = jnp.stack([ktaps(w_max), ktaps(w3), ktaps(w3_)])
    bl = jnp.stack([b_max, b3, b3_]).reshape(3, 1, f)
    m_up = jnp.asarray(np.kron(_bilinear_matrix(H, 9).T,
                               _bilinear_matrix(W, 9).T), jnp.bfloat16)

    out_flat = pl.pallas_call(
        _mega_body,
        out_shape=jax.ShapeDtypeStruct((B, C, N), x.dtype),
        grid=(B,),
        in_specs=[
            pl.BlockSpec((1, C, N), lambda b: (b, 0, 0)),
            pl.BlockSpec((C, f), lambda b: (0, 0)),
            pl.BlockSpec((1, f), lambda b: (0, 0)),
            pl.BlockSpec((9, f, f), lambda b: (0, 0, 0)),
            pl.BlockSpec((1, f), lambda b: (0, 0)),
            pl.BlockSpec((3, 9, f, f), lambda b: (0, 0, 0, 0)),
            pl.BlockSpec((3, 1, f), lambda b: (0, 0, 0)),
            pl.BlockSpec((f, C), lambda b: (0, 0)),
            pl.BlockSpec((f, 1), lambda b: (0, 0)),
            pl.BlockSpec((f, f), lambda b: (0, 0)),
            pl.BlockSpec((f, 1), lambda b: (0, 0)),
            pl.BlockSpec((C, f), lambda b: (0, 0)),
            pl.BlockSpec((C, 1), lambda b: (0, 0)),
            pl.BlockSpec((81, N), lambda b: (0, 0)),
            pl.BlockSpec((792, 81), lambda b: (0, 0)),
        ],
        out_specs=pl.BlockSpec((1, C, N), lambda b: (b, 0, 0)),
        scratch_shapes=[
            pltpu.VMEM((66, 66, f), jnp.float32),
            pltpu.VMEM((3, 66, 16, f), jnp.int32),
            pltpu.VMEM((32, 32, f), jnp.float32),
            pltpu.VMEM((9, 32, f), jnp.float32),
        ],
        compiler_params=pltpu.CompilerParams(
            dimension_semantics=("parallel",),
            vmem_limit_bytes=64 << 20),
    )(x_flat, w1t.astype(bh), b1.reshape(1, f),
      jnp.transpose(w2, (2, 3, 1, 0)).reshape(9, f, f).astype(bh),
      b2.reshape(1, f), wl.astype(bh), bl,
      w1[:, :, 0, 0].astype(bh), b1.reshape(f, 1),
      wf[:, :, 0, 0].astype(bh), bf.reshape(f, 1),
      w4[:, :, 0, 0].astype(bh), b4.reshape(C, 1), m_up,
      jnp.asarray(_shift_gather(9, 9), bh))
    return out_flat.reshape(B, C, H, W)


# speedup vs baseline: 1.0111x; 1.0111x over previous
import numpy as np

import jax
import jax.numpy as jnp
from jax import lax
from jax.experimental import pallas as pl
from jax.experimental.pallas import tpu as pltpu

_TAPS = tuple((dy, dx) for dy in range(3) for dx in range(3))
_CONTRACT0 = (((0,), (0,)), ((), ()))


def _mega_body(x_ref, w1t_ref, b1r_ref, w2t_ref, b2r_ref, wl_ref, bl_ref,
               w1_ref, b1_ref, wf_ref, bf_ref, w4_ref, b4_ref, m_ref, t_ref,
               o_ref, zp_ref, cs_ref, p2_ref, pm_ref):
    f = w1t_ref.shape[1]
    for g in range(x_ref.shape[0]):
        _one_item(g, x_ref, w1t_ref, b1r_ref, w2t_ref, b2r_ref, wl_ref,
                  bl_ref, w1_ref, b1_ref, wf_ref, bf_ref, w4_ref, b4_ref,
                  m_ref, t_ref, o_ref, zp_ref, cs_ref, p2_ref, pm_ref, f)


def _one_item(g, x_ref, w1t_ref, b1r_ref, w2t_ref, b2r_ref, wl_ref, bl_ref,
              w1_ref, b1_ref, wf_ref, bf_ref, w4_ref, b4_ref, m_ref, t_ref,
              o_ref, zp_ref, cs_ref, p2_ref, pm_ref, f):
    x = x_ref[g]
    xb = x.astype(jnp.bfloat16)

    c1t = lax.dot_general(xb, w1t_ref[...], _CONTRACT0,
                          preferred_element_type=jnp.float32) + b1r_ref[...]

    zp_ref[g, 0:1, :, :] = jnp.zeros((1, 66, f), jnp.float32)
    zp_ref[g, :, 0:1, :] = jnp.zeros((66, 1, f), jnp.float32)
    zp_ref[g, 1:65, 1:65, :] = c1t.reshape(64, 64, f)
    for dx in range(3):
        cs_ref[g, dx] = pltpu.bitcast(
            zp_ref[g, :, dx:dx + 63:2, :].astype(jnp.bfloat16), jnp.int32)
    acc = jnp.zeros((1024, f), jnp.float32) + b2r_ref[...]
    for k, (dy, dx) in enumerate(_TAPS):
        tap = pltpu.bitcast(cs_ref[g, dx, dy:dy + 63:2, :, :],
                            jnp.bfloat16)
        acc = acc + jnp.dot(tap.reshape(1024, f), w2t_ref[k],
                            preferred_element_type=jnp.float32)
    p2_ref[g] = acc.reshape(32, 32, f)

    m1 = p2_ref[g, 0:25:3, :, :]
    for ky in range(1, 7):
        m1 = jnp.maximum(m1, p2_ref[g, ky:ky + 25:3, :, :])
    pm_ref[g] = m1
    vm = pm_ref[g, :, 0:25:3, :]
    for kx in range(1, 7):
        vm = jnp.maximum(vm, pm_ref[g, :, kx:kx + 25:3, :])

    z = vm.reshape(81, f).astype(jnp.bfloat16)
    for layer in range(3):
        zs = jnp.dot(t_ref[...], z,
                     preferred_element_type=jnp.float32).astype(jnp.bfloat16)
        acc = jnp.zeros((81, f), jnp.float32) + bl_ref[layer]
        for k in range(9):
            acc = acc + jnp.dot(zs[88 * k:88 * k + 81, :], wl_ref[layer, k],
                                preferred_element_type=jnp.float32)
        if layer < 2:
            acc = jnp.maximum(acc, 0.0)
        z = acc.astype(jnp.bfloat16)
    c3t = z

    up = lax.dot_general(c3t, m_ref[...], _CONTRACT0,
                         preferred_element_type=jnp.float32)

    c1 = jnp.dot(w1_ref[...], xb, preferred_element_type=jnp.float32) + b1_ref[...]
    cf = (jnp.dot(wf_ref[...], c1.astype(jnp.bfloat16),
                  preferred_element_type=jnp.float32) + bf_ref[...] + up)
    c4 = jnp.dot(w4_ref[...], cf.astype(jnp.bfloat16),
                 preferred_element_type=jnp.float32) + b4_ref[...]
    o_ref[g] = x * jax.nn.sigmoid(c4)


def _shift_gather(hm, wm):
    hw = hm * wm
    t = np.zeros((9 * 88, hw), np.float32)
    for k, (dy, dx) in enumerate(_TAPS):
        for i in range(hm):
            si = i + dy - 1
            if si < 0 or si >= hm:
                continue
            for j in range(wm):
                sj = j + dx - 1
                if 0 <= sj < wm:
                    t[88 * k + i * wm + j, si * wm + sj] = 1.0
    return t


def _bilinear_matrix(out_size, in_size):
    scale = in_size / out_size
    dst = np.arange(out_size, dtype=np.float64)
    src = np.clip((dst + 0.5) * scale - 0.5, 0.0, in_size - 1)
    i0 = np.clip(np.floor(src).astype(np.int64), 0, in_size - 1)
    i1 = np.minimum(i0 + 1, in_size - 1)
    w1 = (src - i0).astype(np.float32)
    w0 = 1.0 - w1
    m = np.zeros((out_size, in_size), np.float32)
    rows = np.arange(out_size)
    np.add.at(m, (rows, i0), w0)
    np.add.at(m, (rows, i1), w1)
    return m


def kernel(x, w1, b1, wf, bf, w_max, b_max, w2, b2, w3, b3, w3_, b3_, w4, b4):
    B, C, H, W = x.shape
    N = H * W
    f = w1.shape[0]
    x_flat = x.reshape(B, C, N)
    bh = jnp.bfloat16

    def ktaps(w):
        return jnp.transpose(w, (2, 3, 1, 0)).reshape(9, f, f)

    w1t = jnp.transpose(w1[:, :, 0, 0])
    wl = jnp.stack([ktaps(w_max), ktaps(w3), ktaps(w3_)])
    bl = jnp.stack([b_max, b3, b3_]).reshape(3, 1, f)
    m_up = jnp.asarray(np.kron(_bilinear_matrix(H, 9).T,
                               _bilinear_matrix(W, 9).T), jnp.bfloat16)

    G = 4 if B % 4 == 0 else 1
    out_flat = pl.pallas_call(
        _mega_body,
        out_shape=jax.ShapeDtypeStruct((B, C, N), x.dtype),
        grid=(B // G,),
        in_specs=[
            pl.BlockSpec((G, C, N), lambda b: (b, 0, 0)),
            pl.BlockSpec((C, f), lambda b: (0, 0)),
            pl.BlockSpec((1, f), lambda b: (0, 0)),
            pl.BlockSpec((9, f, f), lambda b: (0, 0, 0)),
            pl.BlockSpec((1, f), lambda b: (0, 0)),
            pl.BlockSpec((3, 9, f, f), lambda b: (0, 0, 0, 0)),
            pl.BlockSpec((3, 1, f), lambda b: (0, 0, 0)),
            pl.BlockSpec((f, C), lambda b: (0, 0)),
            pl.BlockSpec((f, 1), lambda b: (0, 0)),
            pl.BlockSpec((f, f), lambda b: (0, 0)),
            pl.BlockSpec((f, 1), lambda b: (0, 0)),
            pl.BlockSpec((C, f), lambda b: (0, 0)),
            pl.BlockSpec((C, 1), lambda b: (0, 0)),
            pl.BlockSpec((81, N), lambda b: (0, 0)),
            pl.BlockSpec((792, 81), lambda b: (0, 0)),
        ],
        out_specs=pl.BlockSpec((G, C, N), lambda b: (b, 0, 0)),
        scratch_shapes=[
            pltpu.VMEM((G, 66, 66, f), jnp.float32),
            pltpu.VMEM((G, 3, 66, 16, f), jnp.int32),
            pltpu.VMEM((G, 32, 32, f), jnp.float32),
            pltpu.VMEM((G, 9, 32, f), jnp.float32),
        ],
        compiler_params=pltpu.CompilerParams(
            dimension_semantics=("parallel",),
            vmem_limit_bytes=64 << 20),
    )(x_flat, w1t.astype(bh), b1.reshape(1, f),
      jnp.transpose(w2, (2, 3, 1, 0)).reshape(9, f, f).astype(bh),
      b2.reshape(1, f), wl.astype(bh), bl,
      w1[:, :, 0, 0].astype(bh), b1.reshape(f, 1),
      wf[:, :, 0, 0].astype(bh), bf.reshape(f, 1),
      w4[:, :, 0, 0].astype(bh), b4.reshape(C, 1), m_up,
      jnp.asarray(_shift_gather(9, 9), bh))
    return out_flat.reshape(B, C, H, W)
